# baseline (speedup 1.0000x reference)
import jax
import jax.numpy as jnp
from jax.experimental import pallas as pl
from jax.experimental.pallas import tpu as pltpu

_CHUNKS = (512, 1024, 2560, 4096)


def _dma_body(*refs):
    w_hbm, o_hbm = refs[0], refs[1]
    n = len(_CHUNKS)
    bufs = refs[2 : 2 + n]
    rsem, wsem = refs[2 + n], refs[3 + n]
    batch = o_hbm.shape[0]

    offs = [0]
    for rows in _CHUNKS:
        offs.append(offs[-1] + rows)

    reads = [
        pltpu.async_copy(
            w_hbm.at[pl.ds(offs[c], _CHUNKS[c]), :], bufs[c], rsem.at[c]
        )
        for c in range(n)
    ]
    writes = []
    for c in range(n):
        reads[c].wait()
        writes.extend(
            pltpu.async_copy(
                bufs[c], o_hbm.at[b, pl.ds(offs[c], _CHUNKS[c]), :], wsem.at[c]
            )
            for b in range(batch)
        )
    for h in writes:
        h.wait()


def kernel(x, pos_weight):
    batch, seq_len = x.shape
    embed_dim = pos_weight.shape[1]
    assert seq_len == sum(_CHUNKS)

    out = pl.pallas_call(
        _dma_body,
        in_specs=[pl.BlockSpec(memory_space=pl.ANY)],
        out_specs=pl.BlockSpec(memory_space=pl.ANY),
        out_shape=jax.ShapeDtypeStruct((batch, seq_len, embed_dim), pos_weight.dtype),
        scratch_shapes=[
            pltpu.VMEM((rows, embed_dim), pos_weight.dtype) for rows in _CHUNKS
        ]
        + [
            pltpu.SemaphoreType.DMA((len(_CHUNKS),)),
            pltpu.SemaphoreType.DMA((len(_CHUNKS),)),
        ],
    )(pos_weight)
    return out
